# baseline (speedup 1.0000x reference)
import functools

import jax
import jax.numpy as jnp
from jax import lax
from jax.experimental import pallas as pl
from jax.experimental.pallas import tpu as pltpu
from jax.experimental.pallas import tpu_sc as plsc

VOCAB = 1000000
EMBED_DIM = 64
BATCH = 4096
SEQ = 200

NC = 2
NS = 16
NW = NC * NS

ROWS_PER_W = BATCH // NW
ROWS_PER_CHUNK = 2
NBUF = 4
N_CHUNKS = ROWS_PER_W // ROWS_PER_CHUNK
N_STEADY = N_CHUNKS // NBUF - 1

SPLITS = ((0, 128), (128, SEQ - 128))

NCG = VOCAB // 128
TAIL = VOCAB - NCG * 128
C_NBUF = 4
C_FULL = (NCG // NW // C_NBUF) * C_NBUF
C_STEADY = C_FULL // C_NBUF - 1
N_EXTRA = NCG - NW * C_FULL


@functools.partial(
    pl.kernel,
    out_type=jax.ShapeDtypeStruct((VOCAB // 2, 2 * EMBED_DIM), jnp.float32),
    mesh=plsc.VectorSubcoreMesh(core_axis_name="c", subcore_axis_name="s"),
    scratch_types=[
        pltpu.VMEM((C_NBUF, EMBED_DIM, 128), jnp.float32),
        pltpu.VMEM((C_NBUF, EMBED_DIM, 128), jnp.float32),
        pltpu.VMEM((EMBED_DIM, TAIL), jnp.float32),
        [pltpu.SemaphoreType.DMA] * C_NBUF,
        [pltpu.SemaphoreType.DMA] * C_NBUF,
    ],
    compiler_params=pltpu.CompilerParams(needs_layout_passes=False),
)
def _tc_sc(tt_hbm, tail_hbm, out_hbm, vin, vout, vtail, isems, osems):
    wid = lax.axis_index("s") * NC + lax.axis_index("c")

    def start_in(g, b):
        off = pl.multiple_of(g * 128, 128)
        for jr in range(EMBED_DIM // 8):
            pltpu.async_copy(
                tt_hbm.at[pl.ds(jr * 8, 8), pl.ds(off, 128)],
                vin.at[b, pl.ds(jr * 8, 8)],
                isems[b],
            )

    def wait_in(b):
        pltpu.make_async_copy(
            tt_hbm.at[pl.ds(0, EMBED_DIM), pl.ds(0, 128)], vin.at[b], isems[b]
        ).wait()

    lanes = lax.iota(jnp.int32, 16)

    def transpose(src_ref, dst_ref, b, w):
        def rowq(q, carry):
            for v in range(8):
                h = v // 4
                rows = lanes + 16 * (v % 4)
                cols = jnp.full((16,), 0, jnp.int32) + (2 * q + h)
                vals = plsc.load_gather(src_ref, [rows, cols])
                dst_ref[b, q, pl.ds(v * 16, 16)] = vals
            return carry

        lax.fori_loop(0, w // 2, rowq, 0)

    def start_out(g, b, w=128):
        off = pl.multiple_of(g * 64, 8)
        pltpu.async_copy(
            vout.at[b, pl.ds(0, w // 2)],
            out_hbm.at[pl.ds(off, w // 2)],
            osems[b],
        )

    def wait_out(b, w=128):
        pltpu.make_async_copy(
            vout.at[b, pl.ds(0, w // 2)],
            out_hbm.at[pl.ds(0, w // 2)],
            osems[b],
        ).wait()

    for b in range(C_NBUF):
        start_in(b * NW + wid, b)

    def steady(p, carry):
        k0 = p * C_NBUF
        for b in range(C_NBUF):
            wait_in(b)
            transpose(vin.at[b], vout, b, 128)
            start_out((k0 + b) * NW + wid, b)
        for b in range(C_NBUF):
            wait_out(b)
            start_in((k0 + C_NBUF + b) * NW + wid, b)
        return carry

    lax.fori_loop(0, C_STEADY, steady, 0)

    for b in range(C_NBUF):
        wait_in(b)
        transpose(vin.at[b], vout, b, 128)
        start_out((C_FULL - C_NBUF + b) * NW + wid, b)

    for b in range(C_NBUF):
        wait_out(b)

    @pl.when(wid < N_EXTRA)
    def _():
        start_in(NW * C_FULL + wid, 0)
        wait_in(0)
        transpose(vin.at[0], vout, 0, 128)
        start_out(NW * C_FULL + wid, 0)
        wait_out(0)

    @pl.when(wid == N_EXTRA)
    def _():
        pltpu.async_copy(tail_hbm, vtail, isems[0])
        pltpu.make_async_copy(tail_hbm, vtail, isems[0]).wait()
        transpose(vtail, vout, 0, TAIL)
        start_out(NCG, 0, w=TAIL)
        wait_out(0, w=TAIL)


@functools.partial(
    pl.kernel,
    out_type=jax.ShapeDtypeStruct((BATCH * SEQ, 2 * EMBED_DIM), jnp.float32),
    mesh=plsc.VectorSubcoreMesh(core_axis_name="c", subcore_axis_name="s"),
    scratch_types=[
        pltpu.VMEM((ROWS_PER_W, SEQ), jnp.int32),
        pltpu.VMEM((NBUF, ROWS_PER_CHUNK * SEQ, EMBED_DIM), jnp.float32),
        [pltpu.SemaphoreType.DMA] * NBUF,
        [pltpu.SemaphoreType.DMA] * NBUF,
    ],
    compiler_params=pltpu.CompilerParams(use_tc_tiling_on_sc=False),
)
def _gather_sc(table_hbm, idx_hbm, out_hbm, idx_all, rows_v, gsems, wsems):
    wid = lax.axis_index("s") * NC + lax.axis_index("c")
    base = wid * ROWS_PER_W

    pltpu.sync_copy(idx_hbm.at[pl.ds(base, ROWS_PER_W)], idx_all)

    def start_gather(c, b):
        for k in range(ROWS_PER_CHUNK):
            r = c * ROWS_PER_CHUNK + k
            for off, n in SPLITS:
                pltpu.async_copy(
                    table_hbm.at[idx_all.at[r, pl.ds(off, n)]],
                    rows_v.at[b, pl.ds(k * SEQ + off, n)],
                    gsems[b],
                )

    def wait_gather(b):
        pltpu.make_async_copy(
            out_hbm.at[pl.ds(0, ROWS_PER_CHUNK * SEQ), pl.ds(0, EMBED_DIM)],
            rows_v.at[b],
            gsems[b],
        ).wait()

    def start_write(c, b):
        pltpu.async_copy(
            rows_v.at[b],
            out_hbm.at[
                pl.ds((base + c * ROWS_PER_CHUNK) * SEQ, ROWS_PER_CHUNK * SEQ),
                pl.ds(0, EMBED_DIM),
            ],
            wsems[b],
        )

    def wait_write(b):
        pltpu.make_async_copy(
            rows_v.at[b],
            out_hbm.at[pl.ds(0, ROWS_PER_CHUNK * SEQ), pl.ds(0, EMBED_DIM)],
            wsems[b],
        ).wait()

    for b in range(NBUF):
        start_gather(b, b)

    def steady(p, carry):
        c0 = p * NBUF
        for b in range(NBUF):
            wait_gather(b)
            start_write(c0 + b, b)
        for b in range(NBUF):
            wait_write(b)
            start_gather(c0 + NBUF + b, b)
        return carry

    lax.fori_loop(0, N_STEADY, steady, 0)

    for b in range(NBUF):
        wait_gather(b)
        start_write(N_CHUNKS - NBUF + b, b)
    for b in range(NBUF):
        wait_write(b)


def kernel(input, table):
    compact = _tc_sc(table.T, table[VOCAB - TAIL :].T)
    out = _gather_sc(compact.reshape(VOCAB, EMBED_DIM), input)
    return out.reshape(BATCH, SEQ, 2 * EMBED_DIM)[:, :, :EMBED_DIM]


# speedup vs baseline: 2.0708x; 2.0708x over previous
import functools

import jax
import jax.numpy as jnp
from jax import lax
from jax.experimental import pallas as pl
from jax.experimental.pallas import tpu as pltpu
from jax.experimental.pallas import tpu_sc as plsc

VOCAB = 1000000
EMBED_DIM = 64
BATCH = 4096
SEQ = 200

NC = 2
NS = 16
NW = NC * NS

ROWS_PER_W = BATCH // NW
ROWS_PER_CHUNK = 2
NBUF = 4
N_CHUNKS = ROWS_PER_W // ROWS_PER_CHUNK
N_STEADY = N_CHUNKS // NBUF - 1

SPLITS = ((0, 128), (128, SEQ - 128))

@functools.partial(
    pl.kernel,
    out_type=jax.ShapeDtypeStruct((BATCH * SEQ, 2 * EMBED_DIM), jnp.float32),
    mesh=plsc.VectorSubcoreMesh(core_axis_name="c", subcore_axis_name="s"),
    scratch_types=[
        pltpu.VMEM((ROWS_PER_W, SEQ), jnp.int32),
        pltpu.VMEM((NBUF, ROWS_PER_CHUNK * SEQ, EMBED_DIM), jnp.float32),
        [pltpu.SemaphoreType.DMA] * NBUF,
        [pltpu.SemaphoreType.DMA] * NBUF,
    ],
    compiler_params=pltpu.CompilerParams(use_tc_tiling_on_sc=False),
)
def _gather_sc(table_hbm, idx_hbm, out_hbm, idx_all, rows_v, gsems, wsems):
    wid = lax.axis_index("s") * NC + lax.axis_index("c")
    base = wid * ROWS_PER_W

    pltpu.sync_copy(idx_hbm.at[pl.ds(base, ROWS_PER_W)], idx_all)

    def start_gather(c, b):
        for k in range(ROWS_PER_CHUNK):
            r = c * ROWS_PER_CHUNK + k
            for off, n in SPLITS:
                pltpu.async_copy(
                    table_hbm.at[idx_all.at[r, pl.ds(off, n)]],
                    rows_v.at[b, pl.ds(k * SEQ + off, n)],
                    gsems[b],
                )

    def wait_gather(b):
        pltpu.make_async_copy(
            out_hbm.at[pl.ds(0, ROWS_PER_CHUNK * SEQ), pl.ds(0, EMBED_DIM)],
            rows_v.at[b],
            gsems[b],
        ).wait()

    def start_write(c, b):
        pltpu.async_copy(
            rows_v.at[b],
            out_hbm.at[
                pl.ds((base + c * ROWS_PER_CHUNK) * SEQ, ROWS_PER_CHUNK * SEQ),
                pl.ds(0, EMBED_DIM),
            ],
            wsems[b],
        )

    def wait_write(b):
        pltpu.make_async_copy(
            rows_v.at[b],
            out_hbm.at[pl.ds(0, ROWS_PER_CHUNK * SEQ), pl.ds(0, EMBED_DIM)],
            wsems[b],
        ).wait()

    for b in range(NBUF):
        start_gather(b, b)

    def steady(p, carry):
        c0 = p * NBUF
        for b in range(NBUF):
            wait_gather(b)
            start_write(c0 + b, b)
        for b in range(NBUF):
            wait_write(b)
            start_gather(c0 + NBUF + b, b)
        return carry

    lax.fori_loop(0, N_STEADY, steady, 0)

    for b in range(NBUF):
        wait_gather(b)
        start_write(N_CHUNKS - NBUF + b, b)
    for b in range(NBUF):
        wait_write(b)


def kernel(input, table):
    out = _gather_sc(table, input)
    return out.reshape(BATCH, SEQ, 2 * EMBED_DIM)[:, :, :EMBED_DIM]
